# baseline (speedup 1.0000x reference)
import functools

import jax
import jax.numpy as jnp
from jax import lax
from jax.experimental import pallas as pl
from jax.experimental.pallas import tpu as pltpu
from jax.experimental.pallas import tpu_sc as plsc

SEQ = 4096
FEAT = 26
DIM = 64
BATCH = 4
ROWD = FEAT * DIM

NW = 32
ROWS_W = SEQ // NW
CH = 32
NCH = ROWS_W // CH
LANES = ROWD // 16


def _sc_add(x_hbm, t_hbm, o_hbm, xv, tv):
    wid = lax.axis_index("s") * 2 + lax.axis_index("c")
    base = wid * ROWS_W
    for p in range(NCH):
        r0 = base + p * CH
        pltpu.sync_copy(t_hbm.at[pl.ds(r0, CH)], tv)
        for b in range(BATCH):
            pltpu.sync_copy(x_hbm.at[b, pl.ds(r0, CH)], xv)

            def _row(r, carry):
                def _col(c, carry2):
                    s = pl.ds(c * 16, 16)
                    xv[r, s] = xv[r, s] + tv[r, s]
                    return carry2

                return lax.fori_loop(0, LANES, _col, carry)

            lax.fori_loop(0, CH, _row, 0)
            pltpu.sync_copy(xv, o_hbm.at[b, pl.ds(r0, CH)])


def kernel(inputs, table):
    x = inputs.reshape(BATCH, SEQ, ROWD)
    t = table.reshape(SEQ, ROWD)
    sc_call = functools.partial(
        pl.kernel,
        mesh=plsc.VectorSubcoreMesh(core_axis_name="c", subcore_axis_name="s"),
        out_type=jax.ShapeDtypeStruct((BATCH, SEQ, ROWD), jnp.float32),
        scratch_types=[
            pltpu.VMEM((CH, ROWD), jnp.float32),
            pltpu.VMEM((CH, ROWD), jnp.float32),
        ],
    )(_sc_add)
    out = sc_call(x, t)
    return out.reshape(BATCH, SEQ, FEAT, DIM)


# speedup vs baseline: 1.3214x; 1.3214x over previous
import functools

import jax
import jax.numpy as jnp
from jax import lax
from jax.experimental import pallas as pl
from jax.experimental.pallas import tpu as pltpu
from jax.experimental.pallas import tpu_sc as plsc

SEQ = 4096
FEAT = 26
DIM = 64
BATCH = 4
ROWD = FEAT * DIM

NW = 32
ROWS_W = SEQ // NW
CH = 32
NCH = ROWS_W // CH
LANES = ROWD // 16


def _sc_add(x_hbm, t_hbm, o_hbm, xv, tv):
    wid = lax.axis_index("s") * 2 + lax.axis_index("c")
    base = wid * ROWS_W

    def _chunk(p, carry):
        r0 = base + p * CH
        pltpu.sync_copy(t_hbm.at[pl.ds(r0, CH)], tv)

        def _batch(b, carry2):
            pltpu.sync_copy(x_hbm.at[b, pl.ds(r0, CH)], xv)

            def _row(r, carry3):
                for c in range(LANES):
                    s = pl.ds(c * 16, 16)
                    xv[r, s] = xv[r, s] + tv[r, s]
                return carry3

            lax.fori_loop(0, CH, _row, 0)
            pltpu.sync_copy(xv, o_hbm.at[b, pl.ds(r0, CH)])
            return carry2

        return lax.fori_loop(0, BATCH, _batch, carry)

    lax.fori_loop(0, NCH, _chunk, 0)


def kernel(inputs, table):
    x = inputs.reshape(BATCH, SEQ, ROWD)
    t = table.reshape(SEQ, ROWD)
    sc_call = functools.partial(
        pl.kernel,
        mesh=plsc.VectorSubcoreMesh(core_axis_name="c", subcore_axis_name="s"),
        out_type=jax.ShapeDtypeStruct((BATCH, SEQ, ROWD), jnp.float32),
        scratch_types=[
            pltpu.VMEM((CH, ROWD), jnp.float32),
            pltpu.VMEM((CH, ROWD), jnp.float32),
        ],
    )(_sc_add)
    out = sc_call(x, t)
    return out.reshape(BATCH, SEQ, FEAT, DIM)


# speedup vs baseline: 1.4391x; 1.0891x over previous
import functools

import jax
import jax.numpy as jnp
from jax import lax
from jax.experimental import pallas as pl
from jax.experimental.pallas import tpu as pltpu
from jax.experimental.pallas import tpu_sc as plsc

SEQ = 4096
FEAT = 26
DIM = 64
BATCH = 4
ROWD = FEAT * DIM

NW = 32
ROWS_W = SEQ // NW
CH = 16
NCH = ROWS_W // CH
LANES = ROWD // 16


def _sc_add(x_hbm, t_hbm, o_hbm, xv0, xv1, tv, sx, so):
    wid = lax.axis_index("s") * 2 + lax.axis_index("c")
    base = wid * ROWS_W
    bufs = (xv0, xv1)

    def _xcopy(b, r0, i):
        return pltpu.make_async_copy(
            x_hbm.at[b, pl.ds(r0, CH)], bufs[i], sx.at[i]
        )

    def _ocopy(b, r0, i):
        return pltpu.make_async_copy(
            bufs[i], o_hbm.at[b, pl.ds(r0, CH)], so.at[i]
        )

    def _chunk(p, carry):
        r0 = base + p * CH
        pltpu.sync_copy(t_hbm.at[pl.ds(r0, CH)], tv)
        _xcopy(0, r0, 0).start()
        for b in range(BATCH):
            i = b % 2
            _xcopy(b, r0, i).wait()
            if b + 1 < BATCH:
                if b - 1 >= 0:
                    _ocopy(b - 1, r0, (b + 1) % 2).wait()
                _xcopy(b + 1, r0, (b + 1) % 2).start()
            cur = bufs[i]

            def _row(r, carry3, cur=cur):
                for c in range(LANES):
                    s = pl.ds(c * 16, 16)
                    cur[r, s] = cur[r, s] + tv[r, s]
                return carry3

            lax.fori_loop(0, CH, _row, 0)
            _ocopy(b, r0, i).start()
        _ocopy(BATCH - 2, r0, (BATCH - 2) % 2).wait()
        _ocopy(BATCH - 1, r0, (BATCH - 1) % 2).wait()
        return carry

    lax.fori_loop(0, NCH, _chunk, 0)


def kernel(inputs, table):
    x = inputs.reshape(BATCH, SEQ, ROWD)
    t = table.reshape(SEQ, ROWD)
    sc_call = functools.partial(
        pl.kernel,
        mesh=plsc.VectorSubcoreMesh(core_axis_name="c", subcore_axis_name="s"),
        out_type=jax.ShapeDtypeStruct((BATCH, SEQ, ROWD), jnp.float32),
        scratch_types=[
            pltpu.VMEM((CH, ROWD), jnp.float32),
            pltpu.VMEM((CH, ROWD), jnp.float32),
            pltpu.VMEM((CH, ROWD), jnp.float32),
            pltpu.SemaphoreType.DMA((2,)),
            pltpu.SemaphoreType.DMA((2,)),
        ],
    )(_sc_add)
    out = sc_call(x, t)
    return out.reshape(BATCH, SEQ, FEAT, DIM)


# speedup vs baseline: 1.7517x; 1.2173x over previous
import jax
import jax.numpy as jnp
from jax.experimental import pallas as pl
from jax.experimental.pallas import tpu as pltpu

SEQ = 4096
FEAT = 26
DIM = 64
BATCH = 4
ROWD = FEAT * DIM

CH = 128
NSTEP = SEQ // CH
DEPTH = 6


def _x_copy(x_hbm, xb, sx, step, slot, k):
    return pltpu.make_async_copy(
        x_hbm.at[k, pl.ds(step * CH, CH)], xb.at[slot, k], sx.at[slot, k]
    )


def _t_copy(t_hbm, tb, st, step, slot):
    return pltpu.make_async_copy(
        t_hbm.at[pl.ds(step * CH, CH)], tb.at[slot], st.at[slot]
    )


def _o_copy(o_hbm, ob, so, step, slot, k):
    return pltpu.make_async_copy(
        ob.at[slot, k], o_hbm.at[k, pl.ds(step * CH, CH)], so.at[slot, k]
    )


def _body(x_hbm, t_hbm, o_hbm, xb, tb, ob, sx, st, so):
    i = pl.program_id(0)
    slot = jax.lax.rem(i, DEPTH)

    def start_in(step, slot_):
        for k in range(BATCH):
            _x_copy(x_hbm, xb, sx, step, slot_, k).start()
        _t_copy(t_hbm, tb, st, step, slot_).start()

    @pl.when(i == 0)
    def _():
        for d in range(DEPTH - 1):
            start_in(d, d)

    @pl.when(i + DEPTH - 1 < NSTEP)
    def _():
        start_in(i + DEPTH - 1, jax.lax.rem(i + DEPTH - 1, DEPTH))

    for k in range(BATCH):
        _x_copy(x_hbm, xb, sx, i, slot, k).wait()
    _t_copy(t_hbm, tb, st, i, slot).wait()

    @pl.when(i >= DEPTH)
    def _():
        for k in range(BATCH):
            _o_copy(o_hbm, ob, so, i - DEPTH, slot, k).wait()

    t_val = tb[slot]
    for k in range(BATCH):
        ob[slot, k] = xb[slot, k] + t_val
    for k in range(BATCH):
        _o_copy(o_hbm, ob, so, i, slot, k).start()

    @pl.when(i == NSTEP - 1)
    def _():
        for d in range(DEPTH):
            s_ = jax.lax.rem(i - d, DEPTH)
            for k in range(BATCH):
                _o_copy(o_hbm, ob, so, i - d, s_, k).wait()


def kernel(inputs, table):
    x = inputs.reshape(BATCH, SEQ, ROWD)
    t = table.reshape(SEQ, ROWD)
    out = pl.pallas_call(
        _body,
        grid=(NSTEP,),
        in_specs=[
            pl.BlockSpec(memory_space=pl.ANY),
            pl.BlockSpec(memory_space=pl.ANY),
        ],
        out_specs=pl.BlockSpec(memory_space=pl.ANY),
        out_shape=jax.ShapeDtypeStruct((BATCH, SEQ, ROWD), jnp.float32),
        scratch_shapes=[
            pltpu.VMEM((DEPTH, BATCH, CH, ROWD), jnp.float32),
            pltpu.VMEM((DEPTH, CH, ROWD), jnp.float32),
            pltpu.VMEM((DEPTH, BATCH, CH, ROWD), jnp.float32),
            pltpu.SemaphoreType.DMA((DEPTH, BATCH)),
            pltpu.SemaphoreType.DMA((DEPTH,)),
            pltpu.SemaphoreType.DMA((DEPTH, BATCH)),
        ],
        compiler_params=pltpu.CompilerParams(
            dimension_semantics=("arbitrary",),
        ),
    )(x, t)
    return out.reshape(BATCH, SEQ, FEAT, DIM)
